# baseline (speedup 1.0000x reference)
import functools

import jax
import jax.numpy as jnp
from jax import lax
from jax.experimental import pallas as pl
from jax.experimental.pallas import tpu as pltpu
from jax.experimental.pallas import tpu_sc as plsc

_B = 16
_N = 256
_TOTAL = _B * _N
_F = 15
_NC, _NS = 2, 16
_NW = _NC * _NS
_OPW = _TOTAL // _NW
_G = _OPW // 16
_OUT_PW = _OPW * _F

_mesh = plsc.VectorSubcoreMesh(
    core_axis_name="c", subcore_axis_name="s",
    num_cores=_NC, num_subcores=_NS)


@functools.partial(
    pl.kernel,
    out_type=jax.ShapeDtypeStruct((_TOTAL * _F,), jnp.float32),
    mesh=_mesh,
    compiler_params=pltpu.CompilerParams(needs_layout_passes=False),
    scratch_types=[
        pltpu.VMEM((_OPW,), jnp.int32),
        pltpu.VMEM((_OPW,), jnp.int32),
        pltpu.VMEM((_OPW,), jnp.int32),
        pltpu.VMEM((_OPW,), jnp.int32),
        pltpu.VMEM((_B,), jnp.int32),
        pltpu.VMEM((_OUT_PW,), jnp.float32),
    ],
)
def _onehot_sc(col_hbm, mat_hbm, shp_hbm, siz_hbm, len_hbm, out_hbm,
               c_v, m_v, s_v, z_v, len_v, out_v):
    wid = lax.axis_index("s") * _NC + lax.axis_index("c")
    base = wid * _OPW
    pltpu.sync_copy(col_hbm.at[pl.ds(base, _OPW)], c_v)
    pltpu.sync_copy(mat_hbm.at[pl.ds(base, _OPW)], m_v)
    pltpu.sync_copy(shp_hbm.at[pl.ds(base, _OPW)], s_v)
    pltpu.sync_copy(siz_hbm.at[pl.ds(base, _OPW)], z_v)
    pltpu.sync_copy(len_hbm, len_v)

    iota = lax.iota(jnp.int32, 16)
    zeros16 = jnp.zeros((16,), jnp.float32)
    ones16 = jnp.ones((16,), jnp.float32)
    for j in range(_OUT_PW // 16):
        out_v[pl.ds(j * 16, 16)] = zeros16

    scene = wid // 2
    len_scene = plsc.load_gather(len_v, [jnp.full((16,), scene, jnp.int32)])
    halfpos = (wid % 2) * _OPW

    for g in range(_G):
        sl = pl.ds(g * 16, 16)
        pos = halfpos + g * 16 + iota
        ones = jnp.where(pos < len_scene, ones16, zeros16)
        rowbase = (g * 16 + iota) * _F
        plsc.store_scatter(out_v, [rowbase + (c_v[sl] - 10)], ones)
        plsc.store_scatter(out_v, [rowbase + (m_v[sl] - 12)], ones)
        plsc.store_scatter(out_v, [rowbase + (s_v[sl] - 20)], ones)
        plsc.store_scatter(out_v, [rowbase + (z_v[sl] - 27)], ones)

    pltpu.sync_copy(out_v, out_hbm.at[pl.ds(base * _F, _OUT_PW)])


def kernel(input, objects, objects_length, objects_color, objects_material,
           objects_shape, objects_size, relations_spatial_relation):
    flat = _onehot_sc(objects_color, objects_material, objects_shape,
                      objects_size, objects_length)
    obj = flat.reshape(_B, _N, _F)
    rel = relations_spatial_relation.reshape(_B, _N, _N, 4)
    return (obj, rel)
